# baseline (speedup 1.0000x reference)
import functools

import jax
import jax.numpy as jnp
from jax import lax
from jax.experimental import pallas as pl
from jax.experimental.pallas import tpu as pltpu
from jax.experimental.pallas import tpu_sc as plsc

N = 10000
E = 320000
D = 128

NC = 2
NS = 16
NW = NC * NS
NPAD = 10240
EC = 128
NCHUNK = 80
EPW = EC * NCHUNK
EPAD = NW * EPW
RPT = NPAD // NS

CH0 = 132
CH1 = 28
TOTC = NS * (CH0 + CH1)

_sc_mesh = plsc.VectorSubcoreMesh(
    core_axis_name="c", subcore_axis_name="s", num_cores=NC, num_subcores=NS)


@functools.partial(
    pl.kernel,
    out_type=jax.ShapeDtypeStruct((NC * NPAD,), jnp.float32),
    mesh=_sc_mesh,
    scratch_types=[
        pltpu.VMEM((NCHUNK, EC), jnp.int32),
        pltpu.VMEM((EC,), jnp.float32),
        pltpu.VMEM_SHARED((NPAD,), jnp.float32),
        pltpu.SemaphoreType.DMA,
    ],
)
def _sc_degs(dst_hbm, zer_hbm, ones_hbm, out_hbm, didx_t, ones_v, acc_sh, sem):
    c = lax.axis_index("c")
    s = lax.axis_index("s")
    wid = s * NC + c
    pltpu.sync_copy(zer_hbm, acc_sh.at[pl.ds(s * RPT, RPT)])
    pltpu.sync_copy(ones_hbm, ones_v)
    pltpu.sync_copy(dst_hbm.at[wid], didx_t)
    plsc.subcore_barrier()

    def body(j, carry):
        descs = [
            pltpu.async_copy(ones_v, acc_sh.at[didx_t.at[j * 8 + t]], sem,
                             add=True)
            for t in range(8)
        ]
        for d in descs:
            d.wait()
        return carry

    lax.fori_loop(0, NCHUNK // 8, body, 0)
    plsc.subcore_barrier()
    pltpu.sync_copy(acc_sh.at[pl.ds(s * RPT, RPT)],
                    out_hbm.at[pl.ds(c * NPAD + s * RPT, RPT)])


@functools.partial(
    pl.kernel,
    out_type=jax.ShapeDtypeStruct((NC * NPAD, D), jnp.float32),
    mesh=_sc_mesh,
    scratch_types=(
        [pltpu.VMEM((2, EC), jnp.int32)] * 4
        + [pltpu.VMEM((EC, D), jnp.float32)] * 2
        + [pltpu.VMEM((64, D), jnp.float32)]
        + [pltpu.VMEM_SHARED((NPAD, D), jnp.float32)]
        + [pltpu.SemaphoreType.DMA] * 8
    ),
)
def _sc_prop(g_hbm, eidx_hbm, out_hbm, *rest):
    ebuf = rest[0:4]
    rows = rest[4:6]
    zbuf = rest[6]
    acc_sh = rest[7]
    isem = rest[8:12]
    gsem = rest[12:14]
    ssem = rest[14:16]
    c = lax.axis_index("c")
    s = lax.axis_index("s")

    def zrow(r, carry):
        for k in range(D // 16):
            zbuf[r, pl.ds(k * 16, 16)] = jnp.zeros((16,), jnp.float32)
        return carry

    lax.fori_loop(0, 64, zrow, 0)
    for k in range(RPT // 64):
        pltpu.sync_copy(zbuf, acc_sh.at[pl.ds(s * RPT + k * 64, 64)])
    plsc.subcore_barrier()

    def _pipeline(nch, base):
        pltpu.sync_copy(eidx_hbm.at[base], ebuf[0])
        pltpu.async_copy(eidx_hbm.at[base + 1], ebuf[1], isem[1])
        pltpu.async_copy(g_hbm.at[ebuf[0].at[0]], rows[0], gsem[0])

        def body(j, carry):
            for t in range(4):
                i = j * 4 + t
                bg, bs = t % 2, (t - 1) % 2
                pltpu.make_async_copy(g_hbm.at[ebuf[t].at[0]], rows[bg],
                                      gsem[bg]).wait()
                if t < 2:
                    pltpu.async_copy(eidx_hbm.at[base + i + 2], ebuf[t + 2],
                                     isem[t + 2])
                else:
                    @pl.when(j < nch // 4 - 1)
                    def _():
                        pltpu.async_copy(eidx_hbm.at[base + i + 2],
                                         ebuf[t - 2], isem[t - 2])
                if t == 0:
                    @pl.when(j > 0)
                    def _():
                        pltpu.make_async_copy(rows[bs],
                                              acc_sh.at[ebuf[3].at[1]],
                                              ssem[bs]).wait()
                else:
                    pltpu.make_async_copy(rows[bs],
                                          acc_sh.at[ebuf[t - 1].at[1]],
                                          ssem[bs]).wait()
                def _start_gather(tn):
                    pltpu.make_async_copy(eidx_hbm.at[base + i + 1], ebuf[tn],
                                          isem[tn]).wait()
                    pltpu.async_copy(g_hbm.at[ebuf[tn].at[0]], rows[bs],
                                     gsem[bs])
                if t < 3:
                    _start_gather(t + 1)
                else:
                    @pl.when(j < nch // 4 - 1)
                    def _():
                        _start_gather(0)
                pltpu.async_copy(rows[bg], acc_sh.at[ebuf[t].at[1]], ssem[bg],
                                 add=True)
            return carry

        lax.fori_loop(0, nch // 4, body, 0)
        pltpu.make_async_copy(rows[1], acc_sh.at[ebuf[3].at[1]],
                              ssem[1]).wait()

    @pl.when(c == 0)
    def _():
        _pipeline(CH0, s * CH0)

    if CH1 > 0:
        @pl.when(c == 1)
        def _():
            _pipeline(CH1, NS * CH0 + s * CH1)

    plsc.subcore_barrier()
    pltpu.sync_copy(acc_sh.at[pl.ds(s * RPT, RPT)],
                    out_hbm.at[pl.ds(c * NPAD + s * RPT, RPT)])


_BM = 1280


def _tc_prep_body(x_ref, w_ref, d0_ref, d1_ref, g0_ref, ri_ref):
    deg = jnp.maximum(d0_ref[...] + d1_ref[...], 1.0)
    norm = lax.rsqrt(deg)
    h0 = lax.dot_general(x_ref[...], w_ref[...],
                         (((1,), (1,)), ((), ())),
                         preferred_element_type=jnp.float32)
    g0_ref[...] = h0 * norm
    ri_ref[...] = h0 * (norm * norm)


_NBLK = NPAD // _BM

_tc_prep = pl.pallas_call(
    _tc_prep_body,
    grid=(NPAD // _BM,),
    in_specs=[
        pl.BlockSpec((_BM, D), lambda i: (i, 0)),
        pl.BlockSpec((D, D), lambda i: (0, 0)),
        pl.BlockSpec((_BM, 1), lambda i: (i, 0)),
        pl.BlockSpec((_BM, 1), lambda i: (i + _NBLK, 0)),
    ],
    out_specs=[
        pl.BlockSpec((_BM, D), lambda i: (i, 0)),
        pl.BlockSpec((_BM, D), lambda i: (i, 0)),
    ],
    out_shape=[
        jax.ShapeDtypeStruct((NPAD, D), jnp.float32),
        jax.ShapeDtypeStruct((NPAD, D), jnp.float32),
    ],
)


def _tc_comb_body(p0_ref, p1_ref, ri_ref, d0_ref, d1_ref, h_ref, g_ref):
    deg = jnp.maximum(d0_ref[...] + d1_ref[...], 1.0)
    norm = lax.rsqrt(deg)
    h = norm * (p0_ref[...] + p1_ref[...]) + ri_ref[...]
    h_ref[...] = h
    g_ref[...] = h * norm


_tc_comb = pl.pallas_call(
    _tc_comb_body,
    grid=(NPAD // _BM,),
    in_specs=[
        pl.BlockSpec((_BM, D), lambda i: (i, 0)),
        pl.BlockSpec((_BM, D), lambda i: (i + _NBLK, 0)),
        pl.BlockSpec((_BM, D), lambda i: (i, 0)),
        pl.BlockSpec((_BM, 1), lambda i: (i, 0)),
        pl.BlockSpec((_BM, 1), lambda i: (i + _NBLK, 0)),
    ],
    out_specs=[
        pl.BlockSpec((_BM, D), lambda i: (i, 0)),
        pl.BlockSpec((_BM, D), lambda i: (i, 0)),
    ],
    out_shape=[
        jax.ShapeDtypeStruct((NPAD, D), jnp.float32),
        jax.ShapeDtypeStruct((NPAD, D), jnp.float32),
    ],
)


def kernel(features, edge_index, W):
    src = edge_index[0].astype(jnp.int32)
    dst = edge_index[1].astype(jnp.int32)
    pad_e = EPAD - E
    pad_idx = jnp.full((pad_e,), NPAD - 1, dtype=jnp.int32)
    srcp = jnp.concatenate([src, pad_idx]).reshape(TOTC, 1, EC)
    dstp = jnp.concatenate([dst, pad_idx]).reshape(TOTC, 1, EC)
    eidx = jnp.concatenate([srcp, dstp], axis=1)
    xp = jnp.zeros((NPAD, D), jnp.float32).at[:N].set(features)
    zer1 = jnp.zeros((RPT,), jnp.float32)
    ones1 = jnp.ones((EC,), jnp.float32)

    d = _sc_degs(dstp.reshape(NW, NCHUNK, EC), zer1, ones1).reshape(NC * NPAD, 1)
    g0, ri = _tc_prep(xp, W, d, d)
    p = _sc_prop(g0, eidx)
    h1, g1 = _tc_comb(p, p, ri, d, d)
    p2 = _sc_prop(g1, eidx)
    h2, _ = _tc_comb(p2, p2, ri, d, d)
    return h2[:N]


# speedup vs baseline: 1.0989x; 1.0989x over previous
import functools

import jax
import jax.numpy as jnp
from jax import lax
from jax.experimental import pallas as pl
from jax.experimental.pallas import tpu as pltpu
from jax.experimental.pallas import tpu_sc as plsc

N = 10000
E = 320000
D = 128

NC = 2
NS = 16
NW = NC * NS
NPAD = 10240
EC = 128
NCHUNK = 80
EPW = EC * NCHUNK
EPAD = NW * EPW
RPT = NPAD // NS

CH0 = 140
CH1 = 20
TOTC = NS * (CH0 + CH1)

_sc_mesh = plsc.VectorSubcoreMesh(
    core_axis_name="c", subcore_axis_name="s", num_cores=NC, num_subcores=NS)


@functools.partial(
    pl.kernel,
    out_type=jax.ShapeDtypeStruct((NC * NPAD,), jnp.float32),
    mesh=_sc_mesh,
    scratch_types=[
        pltpu.VMEM((NCHUNK, EC), jnp.int32),
        pltpu.VMEM((EC,), jnp.float32),
        pltpu.VMEM_SHARED((NPAD,), jnp.float32),
        pltpu.SemaphoreType.DMA,
    ],
)
def _sc_degs(dst_hbm, zer_hbm, ones_hbm, out_hbm, didx_t, ones_v, acc_sh, sem):
    c = lax.axis_index("c")
    s = lax.axis_index("s")
    wid = s * NC + c
    pltpu.sync_copy(zer_hbm, acc_sh.at[pl.ds(s * RPT, RPT)])
    pltpu.sync_copy(ones_hbm, ones_v)
    pltpu.sync_copy(dst_hbm.at[wid], didx_t)
    plsc.subcore_barrier()

    def body(j, carry):
        descs = [
            pltpu.async_copy(ones_v, acc_sh.at[didx_t.at[j * 8 + t]], sem,
                             add=True)
            for t in range(8)
        ]
        for d in descs:
            d.wait()
        return carry

    lax.fori_loop(0, NCHUNK // 8, body, 0)
    plsc.subcore_barrier()
    pltpu.sync_copy(acc_sh.at[pl.ds(s * RPT, RPT)],
                    out_hbm.at[pl.ds(c * NPAD + s * RPT, RPT)])


@functools.partial(
    pl.kernel,
    out_type=jax.ShapeDtypeStruct((NC * NPAD, D), jnp.float32),
    mesh=_sc_mesh,
    scratch_types=(
        [pltpu.VMEM((2, EC), jnp.int32)] * 4
        + [pltpu.VMEM((EC, D), jnp.float32)] * 2
        + [pltpu.VMEM_SHARED((NPAD, D), jnp.float32)]
        + [pltpu.SemaphoreType.DMA] * 8
    ),
)
def _sc_prop(g_hbm, eidx_hbm, zrows_hbm, out_hbm, *rest):
    ebuf = rest[0:4]
    rows = rest[4:6]
    acc_sh = rest[6]
    isem = rest[7:11]
    gsem = rest[11:13]
    ssem = rest[13:15]
    c = lax.axis_index("c")
    s = lax.axis_index("s")
    pltpu.sync_copy(zrows_hbm, acc_sh.at[pl.ds(s * RPT, RPT)])
    plsc.subcore_barrier()

    def _pipeline(nch, base):
        pltpu.sync_copy(eidx_hbm.at[base], ebuf[0])
        pltpu.async_copy(eidx_hbm.at[base + 1], ebuf[1], isem[1])
        pltpu.async_copy(g_hbm.at[ebuf[0].at[0]], rows[0], gsem[0])

        def body(j, carry):
            for t in range(4):
                i = j * 4 + t
                bg, bs = t % 2, (t - 1) % 2
                pltpu.make_async_copy(g_hbm.at[ebuf[t].at[0]], rows[bg],
                                      gsem[bg]).wait()
                if t < 2:
                    pltpu.async_copy(eidx_hbm.at[base + i + 2], ebuf[t + 2],
                                     isem[t + 2])
                else:
                    @pl.when(j < nch // 4 - 1)
                    def _():
                        pltpu.async_copy(eidx_hbm.at[base + i + 2],
                                         ebuf[t - 2], isem[t - 2])
                if t == 0:
                    @pl.when(j > 0)
                    def _():
                        pltpu.make_async_copy(rows[bs],
                                              acc_sh.at[ebuf[3].at[1]],
                                              ssem[bs]).wait()
                else:
                    pltpu.make_async_copy(rows[bs],
                                          acc_sh.at[ebuf[t - 1].at[1]],
                                          ssem[bs]).wait()
                def _start_gather(tn):
                    pltpu.make_async_copy(eidx_hbm.at[base + i + 1], ebuf[tn],
                                          isem[tn]).wait()
                    pltpu.async_copy(g_hbm.at[ebuf[tn].at[0]], rows[bs],
                                     gsem[bs])
                if t < 3:
                    _start_gather(t + 1)
                else:
                    @pl.when(j < nch // 4 - 1)
                    def _():
                        _start_gather(0)
                pltpu.async_copy(rows[bg], acc_sh.at[ebuf[t].at[1]], ssem[bg],
                                 add=True)
            return carry

        lax.fori_loop(0, nch // 4, body, 0)
        pltpu.make_async_copy(rows[1], acc_sh.at[ebuf[3].at[1]],
                              ssem[1]).wait()

    @pl.when(c == 0)
    def _():
        _pipeline(CH0, s * CH0)

    if CH1 > 0:
        @pl.when(c == 1)
        def _():
            _pipeline(CH1, NS * CH0 + s * CH1)

    plsc.subcore_barrier()
    pltpu.sync_copy(acc_sh.at[pl.ds(s * RPT, RPT)],
                    out_hbm.at[pl.ds(c * NPAD + s * RPT, RPT)])


_BM = 1280


def _tc_prep_body(x_ref, w_ref, d0_ref, d1_ref, g0_ref, ri_ref):
    deg = jnp.maximum(d0_ref[...] + d1_ref[...], 1.0)
    norm = lax.rsqrt(deg)
    h0 = lax.dot_general(x_ref[...], w_ref[...],
                         (((1,), (1,)), ((), ())),
                         preferred_element_type=jnp.float32)
    g0_ref[...] = h0 * norm
    ri_ref[...] = h0 * (norm * norm)


_tc_prep = pl.pallas_call(
    _tc_prep_body,
    grid=(NPAD // _BM,),
    in_specs=[
        pl.BlockSpec((_BM, D), lambda i: (i, 0)),
        pl.BlockSpec((D, D), lambda i: (0, 0)),
        pl.BlockSpec((_BM, 1), lambda i: (i, 0)),
        pl.BlockSpec((_BM, 1), lambda i: (i, 0)),
    ],
    out_specs=[
        pl.BlockSpec((_BM, D), lambda i: (i, 0)),
        pl.BlockSpec((_BM, D), lambda i: (i, 0)),
    ],
    out_shape=[
        jax.ShapeDtypeStruct((NPAD, D), jnp.float32),
        jax.ShapeDtypeStruct((NPAD, D), jnp.float32),
    ],
)


def _tc_comb_body(p0_ref, p1_ref, ri_ref, d0_ref, d1_ref, h_ref, g_ref):
    deg = jnp.maximum(d0_ref[...] + d1_ref[...], 1.0)
    norm = lax.rsqrt(deg)
    h = norm * (p0_ref[...] + p1_ref[...]) + ri_ref[...]
    h_ref[...] = h
    g_ref[...] = h * norm


_tc_comb = pl.pallas_call(
    _tc_comb_body,
    grid=(NPAD // _BM,),
    in_specs=[
        pl.BlockSpec((_BM, D), lambda i: (i, 0)),
        pl.BlockSpec((_BM, D), lambda i: (i, 0)),
        pl.BlockSpec((_BM, D), lambda i: (i, 0)),
        pl.BlockSpec((_BM, 1), lambda i: (i, 0)),
        pl.BlockSpec((_BM, 1), lambda i: (i, 0)),
    ],
    out_specs=[
        pl.BlockSpec((_BM, D), lambda i: (i, 0)),
        pl.BlockSpec((_BM, D), lambda i: (i, 0)),
    ],
    out_shape=[
        jax.ShapeDtypeStruct((NPAD, D), jnp.float32),
        jax.ShapeDtypeStruct((NPAD, D), jnp.float32),
    ],
)


def kernel(features, edge_index, W):
    src = edge_index[0].astype(jnp.int32)
    dst = edge_index[1].astype(jnp.int32)
    pad_e = EPAD - E
    pad_idx = jnp.full((pad_e,), NPAD - 1, dtype=jnp.int32)
    srcp = jnp.concatenate([src, pad_idx]).reshape(TOTC, 1, EC)
    dstp = jnp.concatenate([dst, pad_idx]).reshape(TOTC, 1, EC)
    eidx = jnp.concatenate([srcp, dstp], axis=1)
    xp = jnp.zeros((NPAD, D), jnp.float32).at[:N].set(features)
    zer1 = jnp.zeros((RPT,), jnp.float32)
    ones1 = jnp.ones((EC,), jnp.float32)
    zrows = jnp.zeros((RPT, D), jnp.float32)

    degs = _sc_degs(dstp.reshape(NW, NCHUNK, EC), zer1, ones1)
    degs = degs.reshape(NC, NPAD, 1)
    g0, ri = _tc_prep(xp, W, degs[0], degs[1])
    p = _sc_prop(g0, eidx, zrows).reshape(NC, NPAD, D)
    h1, g1 = _tc_comb(p[0], p[1], ri, degs[0], degs[1])
    p2 = _sc_prop(g1, eidx, zrows).reshape(NC, NPAD, D)
    h2, _ = _tc_comb(p2[0], p2[1], ri, degs[0], degs[1])
    return h2[:N]


# speedup vs baseline: 1.1476x; 1.0442x over previous
import functools

import jax
import jax.numpy as jnp
from jax import lax
from jax.experimental import pallas as pl
from jax.experimental.pallas import tpu as pltpu
from jax.experimental.pallas import tpu_sc as plsc

N = 10000
E = 320000
D = 128

NC = 2
NS = 16
NW = NC * NS
NPAD = 10240
EC = 128
NCHUNK = 80
EPW = EC * NCHUNK
EPAD = NW * EPW
RPT = NPAD // NS

CH0 = 148
CH1 = 12
TOTC = NS * (CH0 + CH1)

_sc_mesh = plsc.VectorSubcoreMesh(
    core_axis_name="c", subcore_axis_name="s", num_cores=NC, num_subcores=NS)


@functools.partial(
    pl.kernel,
    out_type=jax.ShapeDtypeStruct((NC * NPAD,), jnp.float32),
    mesh=_sc_mesh,
    scratch_types=[
        pltpu.VMEM((NCHUNK, EC), jnp.int32),
        pltpu.VMEM((EC,), jnp.float32),
        pltpu.VMEM_SHARED((NPAD,), jnp.float32),
        pltpu.SemaphoreType.DMA,
    ],
)
def _sc_degs(dst_hbm, zer_hbm, ones_hbm, out_hbm, didx_t, ones_v, acc_sh, sem):
    c = lax.axis_index("c")
    s = lax.axis_index("s")
    wid = s * NC + c
    pltpu.sync_copy(zer_hbm, acc_sh.at[pl.ds(s * RPT, RPT)])
    pltpu.sync_copy(ones_hbm, ones_v)
    pltpu.sync_copy(dst_hbm.at[wid], didx_t)
    plsc.subcore_barrier()

    def body(j, carry):
        descs = [
            pltpu.async_copy(ones_v, acc_sh.at[didx_t.at[j * 8 + t]], sem,
                             add=True)
            for t in range(8)
        ]
        for d in descs:
            d.wait()
        return carry

    lax.fori_loop(0, NCHUNK // 8, body, 0)
    plsc.subcore_barrier()
    pltpu.sync_copy(acc_sh.at[pl.ds(s * RPT, RPT)],
                    out_hbm.at[pl.ds(c * NPAD + s * RPT, RPT)])


@functools.partial(
    pl.kernel,
    out_type=jax.ShapeDtypeStruct((NC * NPAD, D), jnp.float32),
    mesh=_sc_mesh,
    scratch_types=(
        [pltpu.VMEM((2, EC), jnp.int32)] * 4
        + [pltpu.VMEM((EC, D), jnp.float32)] * 2
        + [pltpu.VMEM_SHARED((NPAD, D), jnp.float32)]
        + [pltpu.SemaphoreType.DMA] * 8
    ),
)
def _sc_prop(g_hbm, eidx_hbm, zrows_hbm, out_hbm, *rest):
    ebuf = rest[0:4]
    rows = rest[4:6]
    acc_sh = rest[6]
    isem = rest[7:11]
    gsem = rest[11:13]
    ssem = rest[13:15]
    c = lax.axis_index("c")
    s = lax.axis_index("s")
    pltpu.sync_copy(zrows_hbm, acc_sh.at[pl.ds(s * RPT, RPT)])
    plsc.subcore_barrier()

    def _pipeline(nch, base):
        pltpu.sync_copy(eidx_hbm.at[base], ebuf[0])
        pltpu.async_copy(eidx_hbm.at[base + 1], ebuf[1], isem[1])
        pltpu.async_copy(g_hbm.at[ebuf[0].at[0]], rows[0], gsem[0])

        def body(j, carry):
            for t in range(4):
                i = j * 4 + t
                bg, bs = t % 2, (t - 1) % 2
                pltpu.make_async_copy(g_hbm.at[ebuf[t].at[0]], rows[bg],
                                      gsem[bg]).wait()
                if t < 2:
                    pltpu.async_copy(eidx_hbm.at[base + i + 2], ebuf[t + 2],
                                     isem[t + 2])
                else:
                    @pl.when(j < nch // 4 - 1)
                    def _():
                        pltpu.async_copy(eidx_hbm.at[base + i + 2],
                                         ebuf[t - 2], isem[t - 2])
                if t == 0:
                    @pl.when(j > 0)
                    def _():
                        pltpu.make_async_copy(rows[bs],
                                              acc_sh.at[ebuf[3].at[1]],
                                              ssem[bs]).wait()
                else:
                    pltpu.make_async_copy(rows[bs],
                                          acc_sh.at[ebuf[t - 1].at[1]],
                                          ssem[bs]).wait()
                def _start_gather(tn):
                    pltpu.make_async_copy(eidx_hbm.at[base + i + 1], ebuf[tn],
                                          isem[tn]).wait()
                    pltpu.async_copy(g_hbm.at[ebuf[tn].at[0]], rows[bs],
                                     gsem[bs])
                if t < 3:
                    _start_gather(t + 1)
                else:
                    @pl.when(j < nch // 4 - 1)
                    def _():
                        _start_gather(0)
                pltpu.async_copy(rows[bg], acc_sh.at[ebuf[t].at[1]], ssem[bg],
                                 add=True)
            return carry

        lax.fori_loop(0, nch // 4, body, 0)
        pltpu.make_async_copy(rows[1], acc_sh.at[ebuf[3].at[1]],
                              ssem[1]).wait()

    @pl.when(c == 0)
    def _():
        _pipeline(CH0, s * CH0)

    if CH1 > 0:
        @pl.when(c == 1)
        def _():
            _pipeline(CH1, NS * CH0 + s * CH1)

    plsc.subcore_barrier()
    pltpu.sync_copy(acc_sh.at[pl.ds(s * RPT, RPT)],
                    out_hbm.at[pl.ds(c * NPAD + s * RPT, RPT)])


_BM = 1280


def _tc_prep_body(x_ref, w_ref, d0_ref, d1_ref, g0_ref, ri_ref):
    deg = jnp.maximum(d0_ref[...] + d1_ref[...], 1.0)
    norm = lax.rsqrt(deg)
    h0 = lax.dot_general(x_ref[...], w_ref[...],
                         (((1,), (1,)), ((), ())),
                         preferred_element_type=jnp.float32)
    g0_ref[...] = h0 * norm
    ri_ref[...] = h0 * (norm * norm)


_tc_prep = pl.pallas_call(
    _tc_prep_body,
    grid=(NPAD // _BM,),
    in_specs=[
        pl.BlockSpec((_BM, D), lambda i: (i, 0)),
        pl.BlockSpec((D, D), lambda i: (0, 0)),
        pl.BlockSpec((_BM, 1), lambda i: (i, 0)),
        pl.BlockSpec((_BM, 1), lambda i: (i, 0)),
    ],
    out_specs=[
        pl.BlockSpec((_BM, D), lambda i: (i, 0)),
        pl.BlockSpec((_BM, D), lambda i: (i, 0)),
    ],
    out_shape=[
        jax.ShapeDtypeStruct((NPAD, D), jnp.float32),
        jax.ShapeDtypeStruct((NPAD, D), jnp.float32),
    ],
)


def _tc_comb_body(p0_ref, p1_ref, ri_ref, d0_ref, d1_ref, h_ref, g_ref):
    deg = jnp.maximum(d0_ref[...] + d1_ref[...], 1.0)
    norm = lax.rsqrt(deg)
    h = norm * (p0_ref[...] + p1_ref[...]) + ri_ref[...]
    h_ref[...] = h
    g_ref[...] = h * norm


_tc_comb = pl.pallas_call(
    _tc_comb_body,
    grid=(NPAD // _BM,),
    in_specs=[
        pl.BlockSpec((_BM, D), lambda i: (i, 0)),
        pl.BlockSpec((_BM, D), lambda i: (i, 0)),
        pl.BlockSpec((_BM, D), lambda i: (i, 0)),
        pl.BlockSpec((_BM, 1), lambda i: (i, 0)),
        pl.BlockSpec((_BM, 1), lambda i: (i, 0)),
    ],
    out_specs=[
        pl.BlockSpec((_BM, D), lambda i: (i, 0)),
        pl.BlockSpec((_BM, D), lambda i: (i, 0)),
    ],
    out_shape=[
        jax.ShapeDtypeStruct((NPAD, D), jnp.float32),
        jax.ShapeDtypeStruct((NPAD, D), jnp.float32),
    ],
)


def kernel(features, edge_index, W):
    src = edge_index[0].astype(jnp.int32)
    dst = edge_index[1].astype(jnp.int32)
    pad_e = EPAD - E
    pad_idx = jnp.full((pad_e,), NPAD - 1, dtype=jnp.int32)
    srcp = jnp.concatenate([src, pad_idx]).reshape(TOTC, 1, EC)
    dstp = jnp.concatenate([dst, pad_idx]).reshape(TOTC, 1, EC)
    eidx = jnp.concatenate([srcp, dstp], axis=1)
    xp = jnp.zeros((NPAD, D), jnp.float32).at[:N].set(features)
    zer1 = jnp.zeros((RPT,), jnp.float32)
    ones1 = jnp.ones((EC,), jnp.float32)
    zrows = jnp.zeros((RPT, D), jnp.float32)

    degs = _sc_degs(dstp.reshape(NW, NCHUNK, EC), zer1, ones1)
    degs = degs.reshape(NC, NPAD, 1)
    g0, ri = _tc_prep(xp, W, degs[0], degs[1])
    p = _sc_prop(g0, eidx, zrows).reshape(NC, NPAD, D)
    h1, g1 = _tc_comb(p[0], p[1], ri, degs[0], degs[1])
    p2 = _sc_prop(g1, eidx, zrows).reshape(NC, NPAD, D)
    h2, _ = _tc_comb(p2[0], p2[1], ri, degs[0], degs[1])
    return h2[:N]
